# Initial kernel scaffold; baseline (speedup 1.0000x reference)
import jax
import jax.numpy as jnp
from jax.experimental import pallas as pl


def kernel(input, table):
    raise NotImplementedError("write your pallas kernel here")


# speedup vs baseline: 1.6741x; 1.6741x over previous
import functools

import jax
import jax.numpy as jnp
from jax import lax
from jax.experimental import pallas as pl
from jax.experimental.pallas import tpu as pltpu
from jax.experimental.pallas import tpu_sc as plsc

_VOCAB = 100000
_D = 64
_S = 200
_B = 1024

_NC = 2
_NS = 16
_NW = _NC * _NS
_ROWS = _B * _S
_RPW = _ROWS // _NW
_CH = 40
_NCH = _RPW // _CH
_SUB = _S // _CH


def _body(table_hbm, idx_hbm, pos_hbm, out_hbm, idx_v, pos_v, rows_v, sem):
    wid = lax.axis_index("s") * _NC + lax.axis_index("c")
    base = wid * _RPW

    pltpu.sync_copy(pos_hbm, pos_v)
    pltpu.sync_copy(idx_hbm.at[wid], idx_v)

    def chunk(c, _):
        pltpu.make_async_copy(table_hbm.at[idx_v.at[c]], rows_v, sem).start()
        sub = c % _SUB
        p0 = sub * _CH
        pltpu.make_async_copy(table_hbm.at[idx_v.at[c]], rows_v, sem).wait()

        def add_row(r, _):
            for j in range(_D // 16):
                sl = pl.ds(j * 16, 16)
                rows_v[r, sl] = rows_v[r, sl] + pos_v[p0 + r, sl]
            return ()

        lax.fori_loop(0, _CH, add_row, ())
        pltpu.sync_copy(rows_v, out_hbm.at[pl.ds(base + c * _CH, _CH)])
        return ()

    lax.fori_loop(0, _NCH, chunk, ())


@jax.jit
def _run(idx, table, pos):
    mesh = plsc.VectorSubcoreMesh(core_axis_name="c", subcore_axis_name="s")
    f = pl.kernel(
        _body,
        out_type=jax.ShapeDtypeStruct((_ROWS, _D), jnp.float32),
        mesh=mesh,
        scratch_types=[
            pltpu.VMEM((_NCH, _CH), jnp.int32),
            pltpu.VMEM((_S, _D), jnp.float32),
            pltpu.VMEM((_CH, _D), jnp.float32),
            pltpu.SemaphoreType.DMA,
        ],
        compiler_params=pltpu.CompilerParams(use_tc_tiling_on_sc=False),
    )
    return f(table, idx, pos)


def _make_pos(len_seq, embedding_dim):
    positions = jnp.arange(0.0, len_seq)[:, None]
    components_even_idx = jnp.arange(0.0, embedding_dim, 2)
    div = 10000.0 ** (components_even_idx / embedding_dim)
    pos = jnp.zeros((len_seq, embedding_dim), dtype=jnp.float32)
    pos = pos.at[:, 1::2].set(jnp.sin(positions / div))
    pos = pos.at[:, 0::2].set(jnp.cos(positions / div))
    return pos


def kernel(input, table):
    pos = _make_pos(_S, _D)
    idx = input.reshape(_NW, _NCH, _CH).astype(jnp.int32)
    out = _run(idx, table, pos)
    return lax.stop_gradient(out.reshape(_B, _S, _D))


# speedup vs baseline: 2.4215x; 1.4464x over previous
import functools

import jax
import jax.numpy as jnp
from jax import lax
from jax.experimental import pallas as pl
from jax.experimental.pallas import tpu as pltpu
from jax.experimental.pallas import tpu_sc as plsc

_VOCAB = 100000
_D = 64
_S = 200
_B = 1024

_NC = 2
_NS = 16
_NW = _NC * _NS
_ROWS = _B * _S
_RPW = _ROWS // _NW
_CH = 128
_NCH = _RPW // _CH
_NB = 5
_LA = 2


def _body(table_hbm, idx_hbm, pos_hbm, out_hbm, idx_v, pos_v, bufs, gsems, ssems):
    wid = lax.axis_index("s") * _NC + lax.axis_index("c")
    base = wid * _RPW

    pltpu.sync_copy(pos_hbm, pos_v)
    pltpu.sync_copy(idx_hbm.at[wid], idx_v)

    def start_gather(c, b):
        pltpu.make_async_copy(table_hbm.at[idx_v.at[c]], bufs[b], gsems[b]).start()

    def step(c, b, issue_next, wait_store):
        nc = c + _LA
        if issue_next:
            nb = (b + _LA) % _NB
            if wait_store:
                pltpu.make_async_copy(
                    bufs[nb], out_hbm.at[pl.ds(0, _CH)], ssems[nb]).wait()
            start_gather(nc, nb)
        pltpu.make_async_copy(
            table_hbm.at[idx_v.at[c]], bufs[b], gsems[b]).wait()

        p0 = lax.rem(c * _CH, _S)
        buf = bufs[b]

        def add_row(r, p):
            for j in range(_D // 16):
                sl = pl.ds(j * 16, 16)
                buf[r, sl] = buf[r, sl] + pos_v[p, sl]
            p = p + 1
            return lax.select(p == _S, 0, p)

        lax.fori_loop(0, _CH, add_row, p0)
        pltpu.make_async_copy(
            buf, out_hbm.at[pl.ds(base + c * _CH, _CH)], ssems[b]).start()

    for c in range(_LA):
        start_gather(c, c % _NB)
    for c in range(_NB - _LA):
        step(c, c % _NB, issue_next=True, wait_store=False)
    ph = _NB - _LA

    def main(g, _):
        c0 = ph + g * _NB
        for b0 in range(_NB):
            step(c0 + b0, (ph + b0) % _NB, issue_next=True, wait_store=True)
        return ()

    n_main = (_NCH - _LA - ph) // _NB
    lax.fori_loop(0, n_main, main, ())
    for c in range(_NCH - _LA, _NCH):
        step(c, c % _NB, issue_next=False, wait_store=False)
    for c in range(_NCH - _NB, _NCH):
        b = c % _NB
        pltpu.make_async_copy(
            bufs[b], out_hbm.at[pl.ds(0, _CH)], ssems[b]).wait()


@jax.jit
def _run(idx, table, pos):
    mesh = plsc.VectorSubcoreMesh(core_axis_name="c", subcore_axis_name="s")
    f = pl.kernel(
        _body,
        out_type=jax.ShapeDtypeStruct((_ROWS, _D), jnp.float32),
        mesh=mesh,
        scratch_types=[
            pltpu.VMEM((_NCH, _CH), jnp.int32),
            pltpu.VMEM((_S, _D), jnp.float32),
            [pltpu.VMEM((_CH, _D), jnp.float32) for _ in range(_NB)],
            [pltpu.SemaphoreType.DMA for _ in range(_NB)],
            [pltpu.SemaphoreType.DMA for _ in range(_NB)],
        ],
        compiler_params=pltpu.CompilerParams(use_tc_tiling_on_sc=False),
    )
    return f(table, idx, pos)


def _make_pos(len_seq, embedding_dim):
    positions = jnp.arange(0.0, len_seq)[:, None]
    components_even_idx = jnp.arange(0.0, embedding_dim, 2)
    div = 10000.0 ** (components_even_idx / embedding_dim)
    pos = jnp.zeros((len_seq, embedding_dim), dtype=jnp.float32)
    pos = pos.at[:, 1::2].set(jnp.sin(positions / div))
    pos = pos.at[:, 0::2].set(jnp.cos(positions / div))
    return pos


def kernel(input, table):
    pos = _make_pos(_S, _D)
    idx = input.reshape(_NW, _NCH, _CH).astype(jnp.int32)
    out = _run(idx, table, pos)
    return lax.stop_gradient(out.reshape(_B, _S, _D))


# speedup vs baseline: 3.2465x; 1.3407x over previous
import functools

import jax
import jax.numpy as jnp
from jax import lax
from jax.experimental import pallas as pl
from jax.experimental.pallas import tpu as pltpu
from jax.experimental.pallas import tpu_sc as plsc

_VOCAB = 100000
_D = 64
_S = 200
_B = 1024
_H = _S // 2

_NC = 2
_NS = 16
_NW = _NC * _NS
_SPW = _B // _NW
_NB = 4
_LA = 2


def _body(table_hbm, idx_hbm, pos_hbm, out_hbm, idx_v, pos_v, bufs, gsems, ssems):
    wid = lax.axis_index("s") * _NC + lax.axis_index("c")
    sbase = wid * _SPW

    pltpu.sync_copy(pos_hbm, pos_v)
    pltpu.sync_copy(idx_hbm.at[wid], idx_v)

    def start_gather(s, b):
        buf = bufs[b]
        for h in range(2):
            pltpu.make_async_copy(
                table_hbm.at[idx_v.at[s, h]],
                buf.at[pl.ds(h * _H, _H)], gsems[b]).start()

    def wait_gather(s, b):
        buf = bufs[b]
        for h in range(2):
            pltpu.make_async_copy(
                table_hbm.at[idx_v.at[s, h]],
                buf.at[pl.ds(h * _H, _H)], gsems[b]).wait()

    def step(s, b, issue_next, wait_store):
        if issue_next:
            nb = (b + _LA) % _NB
            if wait_store:
                pltpu.make_async_copy(bufs[nb], out_hbm.at[0], ssems[nb]).wait()
            start_gather(s + _LA, nb)
        wait_gather(s, b)
        buf = bufs[b]

        def add_rows(r2, _):
            for u in range(2):
                r = r2 * 2 + u
                for j in range(_D // 16):
                    sl = pl.ds(j * 16, 16)
                    buf[r, sl] = buf[r, sl] + pos_v[r, sl]
            return ()

        lax.fori_loop(0, _S // 2, add_rows, ())
        pltpu.make_async_copy(buf, out_hbm.at[sbase + s], ssems[b]).start()

    for s in range(_LA):
        start_gather(s, s % _NB)
    for s in range(_NB - _LA):
        step(s, s % _NB, issue_next=True, wait_store=False)
    ph = _NB - _LA

    def main(g, _):
        s0 = ph + g * _NB
        for b0 in range(_NB):
            step(s0 + b0, (ph + b0) % _NB, issue_next=True, wait_store=True)
        return ()

    n_main = (_SPW - _LA - ph) // _NB
    lax.fori_loop(0, n_main, main, ())
    for s in range(_SPW - _LA, _SPW):
        step(s, s % _NB, issue_next=False, wait_store=False)
    for s in range(_SPW - _NB, _SPW):
        b = s % _NB
        pltpu.make_async_copy(bufs[b], out_hbm.at[0], ssems[b]).wait()


@jax.jit
def _run(idx, table, pos):
    mesh = plsc.VectorSubcoreMesh(core_axis_name="c", subcore_axis_name="s")
    f = pl.kernel(
        _body,
        out_type=jax.ShapeDtypeStruct((_B, _S, _D), jnp.float32),
        mesh=mesh,
        scratch_types=[
            pltpu.VMEM((_SPW, 2, _H), jnp.int32),
            pltpu.VMEM((_S, _D), jnp.float32),
            [pltpu.VMEM((_S, _D), jnp.float32) for _ in range(_NB)],
            [pltpu.SemaphoreType.DMA for _ in range(_NB)],
            [pltpu.SemaphoreType.DMA for _ in range(_NB)],
        ],
        compiler_params=pltpu.CompilerParams(use_tc_tiling_on_sc=False),
    )
    return f(table, idx, pos)


def _make_pos(len_seq, embedding_dim):
    positions = jnp.arange(0.0, len_seq)[:, None]
    components_even_idx = jnp.arange(0.0, embedding_dim, 2)
    div = 10000.0 ** (components_even_idx / embedding_dim)
    pos = jnp.zeros((len_seq, embedding_dim), dtype=jnp.float32)
    pos = pos.at[:, 1::2].set(jnp.sin(positions / div))
    pos = pos.at[:, 0::2].set(jnp.cos(positions / div))
    return pos


def kernel(input, table):
    pos = _make_pos(_S, _D)
    idx = input.reshape(_NW, _SPW, 2, _H).astype(jnp.int32)
    out = _run(idx, table, pos)
    return lax.stop_gradient(out)
